# baseline (speedup 1.0000x reference)
import functools

import jax
import jax.numpy as jnp
from jax import lax
from jax.experimental import pallas as pl
from jax.experimental.pallas import tpu as pltpu
from jax.experimental.pallas import tpu_sc as plsc

B = 8192
D = 1024
V = 200
CH = 32
NBUF = 3


@functools.cache
def _build():
    info = plsc.get_sparse_core_info()
    NC, NS = info.num_cores, info.num_subcores
    NW = NC * NS
    b_per_w = B // NW
    nch = b_per_w // CH
    stage = V * D // NS
    mesh = plsc.VectorSubcoreMesh(core_axis_name="c", subcore_axis_name="s")

    scratch = (
        [pltpu.VMEM_SHARED((V * D,), jnp.float32)]
        + [pltpu.VMEM((b_per_w,), jnp.int32)]
        + [pltpu.VMEM((CH * D,), jnp.float32) for _ in range(NBUF)]
        + [pltpu.SemaphoreType.DMA for _ in range(2 * NBUF)]
    )

    @functools.partial(
        pl.kernel,
        mesh=mesh,
        out_type=jax.ShapeDtypeStruct((B * D,), jnp.float32),
        scratch_types=scratch,
    )
    def gather_kernel(idx_hbm, table_hbm, out_hbm, table_sh, idx_v, *rest):
        rows = rest[:NBUF]
        gsem = rest[NBUF : 2 * NBUF]
        wsem = rest[2 * NBUF :]
        sid = lax.axis_index("s")
        wid = sid * NC + lax.axis_index("c")
        base = wid * b_per_w
        pltpu.sync_copy(idx_hbm.at[pl.ds(base, b_per_w)], idx_v)

        pltpu.sync_copy(
            table_hbm.at[pl.ds(sid * stage, stage)],
            table_sh.at[pl.ds(sid * stage, stage)],
        )
        plsc.subcore_barrier()

        def chunk_gather(c, b):
            def group_copy(h, _):
                v = idx_v[pl.ds(c * CH + h * 16, 16)]
                for i in range(16):
                    r = v[i]
                    pltpu.async_copy(
                        table_sh.at[pl.ds(r * D, D)],
                        rows[b].at[pl.ds((h * 16 + i) * D, D)],
                        gsem[b],
                    )
                return 0

            lax.fori_loop(0, CH // 16, group_copy, 0)
            return pltpu.make_async_copy(
                table_hbm.at[pl.ds(0, CH * D)], rows[b], gsem[b]
            )

        g = [None] * nch
        w = [None] * nch
        for c in range(min(NBUF, nch)):
            g[c] = chunk_gather(c, c)
        for c in range(nch):
            b = c % NBUF
            g[c].wait()
            w[c] = pltpu.async_copy(
                rows[b], out_hbm.at[pl.ds((base + c * CH) * D, CH * D)], wsem[b]
            )
            nxt = c + NBUF
            if nxt < nch:
                w[c].wait()
                g[nxt] = chunk_gather(nxt, b)
        for c in range(max(nch - NBUF, 0), nch):
            w[c].wait()

    return gather_kernel


def kernel(positions, learned_embeddings):
    positions = positions.astype(jnp.int32)
    out = _build()(positions, learned_embeddings.reshape(-1))
    return out.reshape(B, D)


# speedup vs baseline: 1.8827x; 1.8827x over previous
import functools

import jax
import jax.numpy as jnp
from jax import lax
from jax.experimental import pallas as pl
from jax.experimental.pallas import tpu as pltpu
from jax.experimental.pallas import tpu_sc as plsc

B = 8192
D = 1024
V = 200


@functools.cache
def _build():
    info = plsc.get_sparse_core_info()
    NC, NS = info.num_cores, info.num_subcores
    NW = NC * NS
    b_per_w = B // NW
    stage = V * D // NS
    mesh = plsc.VectorSubcoreMesh(core_axis_name="c", subcore_axis_name="s")

    scratch = [
        pltpu.VMEM_SHARED((V * D,), jnp.float32),
        pltpu.VMEM((b_per_w,), jnp.int32),
        pltpu.SemaphoreType.DMA,
    ]

    @functools.partial(
        pl.kernel,
        mesh=mesh,
        out_type=jax.ShapeDtypeStruct((B, D), jnp.float32),
        scratch_types=scratch,
    )
    def gather_kernel(idx_hbm, table_hbm, out_hbm, table_sh, idx_v, wsem):
        sid = lax.axis_index("s")
        wid = sid * NC + lax.axis_index("c")
        base = wid * b_per_w
        pltpu.sync_copy(idx_hbm.at[pl.ds(base, b_per_w)], idx_v)

        pltpu.sync_copy(
            table_hbm.at[pl.ds(sid * stage, stage)],
            table_sh.at[pl.ds(sid * stage, stage)],
        )
        plsc.subcore_barrier()

        def group_copy(h, _):
            v = idx_v[pl.ds(h * 16, 16)]
            for i in range(16):
                r = v[i]
                pltpu.async_copy(
                    table_sh.at[pl.ds(r * D, D)],
                    out_hbm.at[base + h * 16 + i],
                    wsem,
                )
            return 0

        lax.fori_loop(0, b_per_w // 16, group_copy, 0)

        pltpu.make_async_copy(
            out_hbm.at[pl.ds(0, b_per_w)],
            out_hbm.at[pl.ds(base, b_per_w)],
            wsem,
        ).wait()

    return gather_kernel


def kernel(positions, learned_embeddings):
    positions = positions.astype(jnp.int32)
    return _build()(positions, learned_embeddings.reshape(-1))
